# Initial kernel scaffold; baseline (speedup 1.0000x reference)
import jax
import jax.numpy as jnp
from jax.experimental import pallas as pl


def kernel(positions, pos_embeddings):
    raise NotImplementedError("write your pallas kernel here")


# speedup vs baseline: 3.3520x; 3.3520x over previous
import functools

import jax
import jax.numpy as jnp
from jax import lax
from jax.experimental import pallas as pl
from jax.experimental.pallas import tpu as pltpu
from jax.experimental.pallas import tpu_sc as plsc

_EMBED_DIM = 64
_MAX_SEQ_LEN = 512
_NUM_CORES = 2
_NUM_SUBCORES = 16
_NUM_WORKERS = _NUM_CORES * _NUM_SUBCORES
_LANES = 16
_GROUP = 128


@functools.lru_cache(maxsize=None)
def _build_sc_gather(total, embed_dim, max_seq_len):
    per_w = total // _NUM_WORKERS
    n_groups = per_w // _GROUP
    mesh = plsc.VectorSubcoreMesh(
        core_axis_name="c", subcore_axis_name="s",
        num_cores=_NUM_CORES, num_subcores=_NUM_SUBCORES)

    @functools.partial(
        pl.kernel,
        out_type=jax.ShapeDtypeStruct((total, embed_dim), jnp.float32),
        mesh=mesh,
        scratch_types=[
            pltpu.VMEM((per_w,), jnp.float32),
            pltpu.VMEM((n_groups, _GROUP), jnp.int32),
            pltpu.VMEM((2, _GROUP, embed_dim), jnp.float32),
            pltpu.SemaphoreType.DMA,
            pltpu.SemaphoreType.DMA,
        ],
        compiler_params=pltpu.CompilerParams(use_tc_tiling_on_sc=False),
    )
    def sc_kernel(pos_hbm, table_hbm, out_hbm, pos_v, idx_v, rows_v,
                  gsem, osem):
        wid = lax.axis_index("s") * _NUM_CORES + lax.axis_index("c")
        base = wid * per_w

        pltpu.sync_copy(pos_hbm.at[pl.ds(base, per_w)], pos_v)

        scale = jnp.float32(max_seq_len - 1)
        hi = jnp.int32(max_seq_len - 1)

        def idx_body(g, carry):
            for k in range(_GROUP // _LANES):
                p = pos_v[pl.ds(g * _GROUP + k * _LANES, _LANES)]
                iv = (p * scale).astype(jnp.int32)
                iv = jnp.minimum(jnp.maximum(iv, 0), hi)
                idx_v[g, pl.ds(k * _LANES, _LANES)] = iv
            return carry

        lax.fori_loop(0, n_groups, idx_body, 0)

        def gather(g, slot):
            return pltpu.async_copy(
                table_hbm.at[idx_v.at[g]], rows_v.at[slot], gsem)

        gather(0, 0)

        def step(g, carry):
            for slot in range(2):
                gg = g + slot

                @pl.when(gg < n_groups)
                def _():
                    pltpu.make_async_copy(
                        table_hbm.at[idx_v.at[gg]], rows_v.at[slot],
                        gsem).wait()

                @pl.when(gg + 1 < n_groups)
                def _():
                    gather(gg + 1, 1 - slot)

                @pl.when(gg < n_groups)
                def _():
                    pltpu.sync_copy(
                        rows_v.at[slot],
                        out_hbm.at[pl.ds(base + gg * _GROUP, _GROUP)])
            return carry

        lax.fori_loop(0, pl.cdiv(n_groups, 2), lambda g, c: step(g * 2, c), 0)

    return sc_kernel


def kernel(positions, pos_embeddings):
    max_seq_len, embed_dim = pos_embeddings.shape
    b, n, _ = positions.shape
    pos_flat = positions.reshape(b * n)
    sc = _build_sc_gather(b * n, embed_dim, max_seq_len)
    out = sc(pos_flat, pos_embeddings)
    return out.reshape(b, n, embed_dim)


# speedup vs baseline: 3.4546x; 1.0306x over previous
import functools

import jax
import jax.numpy as jnp
from jax import lax
from jax.experimental import pallas as pl
from jax.experimental.pallas import tpu as pltpu
from jax.experimental.pallas import tpu_sc as plsc

_EMBED_DIM = 64
_MAX_SEQ_LEN = 512
_NUM_CORES = 2
_NUM_SUBCORES = 16
_NUM_WORKERS = _NUM_CORES * _NUM_SUBCORES
_LANES = 16
_GROUP = 128
_K = 4


@functools.lru_cache(maxsize=None)
def _build_sc_gather(total, embed_dim, max_seq_len):
    per_w = total // _NUM_WORKERS
    n_groups = per_w // _GROUP
    mesh = plsc.VectorSubcoreMesh(
        core_axis_name="c", subcore_axis_name="s",
        num_cores=_NUM_CORES, num_subcores=_NUM_SUBCORES)

    @functools.partial(
        pl.kernel,
        out_type=jax.ShapeDtypeStruct((total, embed_dim), jnp.float32),
        mesh=mesh,
        scratch_types=[
            pltpu.VMEM((per_w,), jnp.float32),
            pltpu.VMEM((n_groups, _GROUP), jnp.int32),
            pltpu.VMEM((2, _K * _GROUP, embed_dim), jnp.float32),
            pltpu.SemaphoreType.DMA,
            pltpu.SemaphoreType.DMA,
        ],
        compiler_params=pltpu.CompilerParams(use_tc_tiling_on_sc=False),
    )
    def sc_kernel(pos_hbm, table_hbm, out_hbm, pos_v, idx_v, rows_v,
                  gsem, osem):
        wid = lax.axis_index("s") * _NUM_CORES + lax.axis_index("c")
        base = wid * per_w
        n_steps = n_groups // _K
        step_rows = _K * _GROUP

        pltpu.sync_copy(pos_hbm.at[pl.ds(base, per_w)], pos_v)

        scale = jnp.float32(max_seq_len - 1)
        hi = jnp.int32(max_seq_len - 1)

        def idx_body(g, carry):
            for k in range(_GROUP // _LANES):
                p = pos_v[pl.ds(g * _GROUP + k * _LANES, _LANES)]
                iv = (p * scale).astype(jnp.int32)
                iv = jnp.minimum(jnp.maximum(iv, 0), hi)
                idx_v[g, pl.ds(k * _LANES, _LANES)] = iv
            return carry

        lax.fori_loop(0, n_groups, idx_body, 0)

        def fire(m, slot):
            for j in range(_K):
                pltpu.async_copy(
                    table_hbm.at[idx_v.at[m * _K + j]],
                    rows_v.at[slot].at[pl.ds(j * _GROUP, _GROUP)], gsem)

        def drain(m, slot):
            for j in range(_K):
                pltpu.make_async_copy(
                    table_hbm.at[idx_v.at[m * _K + j]],
                    rows_v.at[slot].at[pl.ds(j * _GROUP, _GROUP)],
                    gsem).wait()

        def out_copy(m, slot):
            return pltpu.make_async_copy(
                rows_v.at[slot],
                out_hbm.at[pl.ds(base + m * step_rows, step_rows)], osem)

        fire(0, 0)

        def step(it, carry):
            for s in range(2):
                m = it * 2 + s
                drain(m, s)

                @pl.when(m >= 1)
                def _():
                    out_copy(m - 1, 1 - s).wait()

                @pl.when(m + 1 < n_steps)
                def _():
                    fire(m + 1, 1 - s)

                out_copy(m, s).start()
            return carry

        lax.fori_loop(0, n_steps // 2, step, 0)
        out_copy(n_steps - 1, 1).wait()

    return sc_kernel


def kernel(positions, pos_embeddings):
    max_seq_len, embed_dim = pos_embeddings.shape
    b, n, _ = positions.shape
    pos_flat = positions.reshape(b * n)
    sc = _build_sc_gather(b * n, embed_dim, max_seq_len)
    out = sc(pos_flat, pos_embeddings)
    return out.reshape(b, n, embed_dim)


# speedup vs baseline: 4.9853x; 1.4431x over previous
import functools

import jax
import jax.numpy as jnp
from jax import lax
from jax.experimental import pallas as pl
from jax.experimental.pallas import tpu as pltpu
from jax.experimental.pallas import tpu_sc as plsc

_EMBED_DIM = 64
_MAX_SEQ_LEN = 512
_NUM_CORES = 2
_NUM_SUBCORES = 16
_NUM_WORKERS = _NUM_CORES * _NUM_SUBCORES
_LANES = 16
_GROUP = 128
_K = 4


@functools.lru_cache(maxsize=None)
def _build_sc_gather(total, embed_dim, max_seq_len):
    per_w = total // _NUM_WORKERS
    n_groups = per_w // _GROUP
    mesh = plsc.VectorSubcoreMesh(
        core_axis_name="c", subcore_axis_name="s",
        num_cores=_NUM_CORES, num_subcores=_NUM_SUBCORES)

    @functools.partial(
        pl.kernel,
        out_type=jax.ShapeDtypeStruct((total, embed_dim), jnp.float32),
        mesh=mesh,
        scratch_types=[
            pltpu.VMEM((per_w,), jnp.float32),
            pltpu.VMEM((n_groups, _GROUP), jnp.int32),
            pltpu.VMEM((2, _K * _GROUP, embed_dim), jnp.float32),
            pltpu.VMEM_SHARED((max_seq_len, embed_dim), jnp.float32),
            pltpu.SemaphoreType.DMA,
            pltpu.SemaphoreType.DMA,
        ],
        compiler_params=pltpu.CompilerParams(use_tc_tiling_on_sc=False),
    )
    def sc_kernel(pos_hbm, table_hbm, out_hbm, pos_v, idx_v, rows_v,
                  spm_table, gsem, osem):
        wid = lax.axis_index("s") * _NUM_CORES + lax.axis_index("c")
        base = wid * per_w
        n_steps = n_groups // _K
        step_rows = _K * _GROUP

        @pl.when(lax.axis_index("s") == 0)
        def _():
            pltpu.sync_copy(table_hbm, spm_table)

        pltpu.sync_copy(pos_hbm.at[pl.ds(base, per_w)], pos_v)

        scale = jnp.float32(max_seq_len - 1)
        hi = jnp.int32(max_seq_len - 1)

        def idx_body(g, carry):
            for k in range(_GROUP // _LANES):
                p = pos_v[pl.ds(g * _GROUP + k * _LANES, _LANES)]
                iv = (p * scale).astype(jnp.int32)
                iv = jnp.minimum(jnp.maximum(iv, 0), hi)
                idx_v[g, pl.ds(k * _LANES, _LANES)] = iv
            return carry

        lax.fori_loop(0, n_groups, idx_body, 0)
        plsc.subcore_barrier()

        def fire(m, slot):
            for j in range(_K):
                pltpu.async_copy(
                    spm_table.at[idx_v.at[m * _K + j]],
                    rows_v.at[slot].at[pl.ds(j * _GROUP, _GROUP)], gsem)

        def drain(m, slot):
            for j in range(_K):
                pltpu.make_async_copy(
                    spm_table.at[idx_v.at[m * _K + j]],
                    rows_v.at[slot].at[pl.ds(j * _GROUP, _GROUP)],
                    gsem).wait()

        def out_copy(m, slot):
            return pltpu.make_async_copy(
                rows_v.at[slot],
                out_hbm.at[pl.ds(base + m * step_rows, step_rows)], osem)

        fire(0, 0)

        def step(it, carry):
            for s in range(2):
                m = it * 2 + s
                drain(m, s)

                @pl.when(m >= 1)
                def _():
                    out_copy(m - 1, 1 - s).wait()

                @pl.when(m + 1 < n_steps)
                def _():
                    fire(m + 1, 1 - s)

                out_copy(m, s).start()
            return carry

        lax.fori_loop(0, n_steps // 2, step, 0)
        out_copy(n_steps - 1, 1).wait()

    return sc_kernel


def kernel(positions, pos_embeddings):
    max_seq_len, embed_dim = pos_embeddings.shape
    b, n, _ = positions.shape
    pos_flat = positions.reshape(b * n)
    sc = _build_sc_gather(b * n, embed_dim, max_seq_len)
    out = sc(pos_flat, pos_embeddings)
    return out.reshape(b, n, embed_dim)
